# Initial kernel scaffold; baseline (speedup 1.0000x reference)
import jax
import jax.numpy as jnp
from jax.experimental import pallas as pl


def kernel(x, edge_index, batch, params):
    raise NotImplementedError("write your pallas kernel here")


# speedup vs baseline: 1.0681x; 1.0681x over previous
import functools

import jax
import jax.numpy as jnp
from jax.experimental import pallas as pl
from jax.experimental.pallas import tpu as pltpu

N = 10000
E = 320000
D = 128
G = 16


def _matmul_body(x_ref, w_ref, b_ref, o_ref):
    o_ref[...] = (
        jnp.dot(x_ref[...], w_ref[...], preferred_element_type=jnp.float32)
        + b_ref[...]
    )


def _dense(x, W, b):
    M, K = x.shape
    F = W.shape[0]
    Wt = W.T
    bm = 2000
    grid = M // bm
    return pl.pallas_call(
        _matmul_body,
        grid=(grid,),
        in_specs=[
            pl.BlockSpec((bm, K), lambda i: (i, 0)),
            pl.BlockSpec((K, F), lambda i: (0, 0)),
            pl.BlockSpec((1, F), lambda i: (0, 0)),
        ],
        out_specs=pl.BlockSpec((bm, F), lambda i: (i, 0)),
        out_shape=jax.ShapeDtypeStruct((M, F), jnp.float32),
    )(x, Wt, b[None, :])


def _conv(x, src, dst, p, l):
    h = _dense(x, p[f'lin_W{l}'], p[f'lin_b{l}'])
    n = h.shape[0]
    x_j = h[src]
    x_i = h[dst]
    a = _dense(jnp.concatenate([x_i, x_j], axis=-1), p[f'att1_W{l}'], p[f'att1_b{l}'])
    a = jnp.where(a > 0, a, 0.1 * a)
    a = (a @ p[f'att2_W{l}'].T + p[f'att2_b{l}'])[:, 0]
    m = jax.ops.segment_max(a, dst, num_segments=n)
    m = jnp.where(jnp.isfinite(m), m, 0.0)
    e = jnp.exp(a - m[dst])
    s = jax.ops.segment_sum(e, dst, num_segments=n)
    w = e / jnp.maximum(s[dst], 1e-16)
    att_out = jax.ops.segment_sum(w[:, None] * x_j, dst, num_segments=n)
    cnt = jax.ops.segment_sum(jnp.ones((src.shape[0],), x.dtype), dst, num_segments=n)
    mean_out = jax.ops.segment_sum(x_j, dst, num_segments=n) / jnp.maximum(cnt, 1.0)[:, None]
    mx = jax.ops.segment_max(x_j, dst, num_segments=n)
    max_out = jnp.where(jnp.isfinite(mx), mx, 0.0)
    cat = jnp.concatenate([mean_out, max_out, att_out], axis=-1)
    g = jnp.maximum(_dense(cat, p[f'gate1_W{l}'], p[f'gate1_b{l}']), 0.0)
    g = g @ p[f'gate2_W{l}'].T + p[f'gate2_b{l}']
    gw = jax.nn.softmax(g, axis=-1)
    fused = gw[:, 0:1] * mean_out + gw[:, 1:2] * max_out + gw[:, 2:3] * att_out
    return fused + h


def _gnorm(x, batch, p, l):
    cnt = jnp.maximum(jax.ops.segment_sum(jnp.ones((x.shape[0],), x.dtype), batch, num_segments=G), 1.0)[:, None]
    mean = jax.ops.segment_sum(x, batch, num_segments=G) / cnt
    out = x - p[f'gn_a{l}'] * mean[batch]
    var = jax.ops.segment_sum(out * out, batch, num_segments=G) / cnt
    std = jnp.sqrt(var + 1e-5)[batch]
    return p[f'gn_w{l}'] * out / std + p[f'gn_b{l}']


def kernel(x, edge_index, batch, params):
    src = edge_index[0]
    dst = edge_index[1]
    h = x
    for l in range(3):
        h = _conv(h, src, dst, params, l)
        h = _gnorm(h, batch, params, l)
        h = jnp.maximum(h, 0.0)
    return _conv(h, src, dst, params, 3)


# speedup vs baseline: 6.2690x; 5.8695x over previous
import functools

import jax
import jax.numpy as jnp
from jax import lax
from jax.experimental import pallas as pl
from jax.experimental.pallas import tpu as pltpu
from jax.experimental.pallas import tpu_sc as plsc

N = 10000
E = 320000
D = 128
G = 16

NT = 32
NB = 313
NPAD = NT * NB
B = 128
BN = 2504
NBLK = NPAD // BN
GP = 32
NEG = -1e30


def _pre_body(x_ref, linWT_ref, linb_ref, wiT_ref, wjT_ref, b1_ref,
              h_ref, haj_ref, ai_ref):
    h = jnp.dot(x_ref[...], linWT_ref[...], preferred_element_type=jnp.float32)
    h = h + linb_ref[...]
    h_ref[...] = h
    haj_ref[:, :D] = h
    haj_ref[:, D:] = jnp.dot(h, wjT_ref[...], preferred_element_type=jnp.float32)
    ai_ref[...] = (
        jnp.dot(h, wiT_ref[...], preferred_element_type=jnp.float32) + b1_ref[...]
    )


def _pre(x, linWT, linb, wiT, wjT, b1):
    return pl.pallas_call(
        _pre_body,
        grid=(NBLK,),
        in_specs=[
            pl.BlockSpec((BN, D), lambda i: (i, 0)),
            pl.BlockSpec((D, D), lambda i: (0, 0)),
            pl.BlockSpec((1, D), lambda i: (0, 0)),
            pl.BlockSpec((D, D), lambda i: (0, 0)),
            pl.BlockSpec((D, D), lambda i: (0, 0)),
            pl.BlockSpec((1, D), lambda i: (0, 0)),
        ],
        out_specs=[
            pl.BlockSpec((BN, D), lambda i: (i, 0)),
            pl.BlockSpec((BN, 2 * D), lambda i: (i, 0)),
            pl.BlockSpec((BN, D), lambda i: (i, 0)),
        ],
        out_shape=[
            jax.ShapeDtypeStruct((NPAD, D), jnp.float32),
            jax.ShapeDtypeStruct((NPAD, 2 * D), jnp.float32),
            jax.ShapeDtypeStruct((NPAD, D), jnp.float32),
        ],
    )(x, linWT, linb, wiT, wjT, b1)


def _edge_body(haj_hbm, ai_hbm, srcs_hbm, dsts_hbm, starts_hbm, w2_hbm,
               out_hbm, starts_v, w2_v, ai_loc, idx_v, dst_v, buf, stage, sem):
    wid = lax.axis_index("s") * 2 + lax.axis_index("c")
    n0 = wid * NB
    pltpu.sync_copy(starts_hbm, starts_v)
    pltpu.sync_copy(w2_hbm, w2_v)
    pltpu.sync_copy(ai_hbm.at[pl.ds(n0 * D, NB * D)], ai_loc)

    sv = starts_v[pl.ds(wid, 16)]
    lo = sv[0]
    hi = sv[1]
    b0 = lo // B
    nbat = jnp.where(hi > lo, (hi - b0 * B + B - 1) // B, 0)

    zero = jnp.zeros((16,), jnp.float32)
    neg = jnp.full((16,), NEG, jnp.float32)

    def flush_upto(cur, cnt, s_vec, att, ssum, smax, d_eff):
        cnt_f = jnp.full((16,), cnt.astype(jnp.float32), jnp.float32)
        has = jnp.minimum(cnt_f, 1.0)
        inv_c = 1.0 / jnp.maximum(cnt_f, 1.0)
        inv_s = 1.0 / jnp.maximum(s_vec, 1e-16)
        cur0 = cur

        def wr(k, _):
            mf = jnp.full((16,), jnp.where(k == cur0, 1.0, 0.0), jnp.float32)
            for j in range(8):
                sl = pl.ds(j * 16, 16)
                stage[sl] = ssum[j] * inv_c * mf
                stage[pl.ds(D + j * 16, 16)] = smax[j] * has * mf
                stage[pl.ds(2 * D + j * 16, 16)] = att[j] * inv_s * mf
            pltpu.sync_copy(stage, out_hbm.at[pl.ds(k * 3 * D, 3 * D)])
            return 0

        lax.fori_loop(cur, d_eff, wr, 0)
        did = d_eff > cur
        kf = jnp.full((16,), jnp.where(did, 0.0, 1.0), jnp.float32)
        cnt = jnp.where(did, jnp.int32(0), cnt)
        s_vec = s_vec * kf
        att = tuple(a * kf for a in att)
        ssum = tuple(a * kf for a in ssum)
        smax = tuple(a * kf + (1.0 - kf) * NEG for a in smax)
        return jnp.maximum(cur, d_eff), cnt, s_vec, att, ssum, smax

    def edge_step(e, carry):
        cur, cnt, s_vec, att, ssum, smax, g0 = carry
        g = g0 + e
        act = jnp.logical_and(g >= lo, g < hi)
        d = dst_v[pl.ds(e, 16)][0]
        d_eff = jnp.where(act, d, cur)
        cur, cnt, s_vec, att, ssum, smax = flush_upto(
            cur, cnt, s_vec, att, ssum, smax, d_eff)
        dl = d_eff - n0
        m = jnp.full((16,), jnp.where(act, 1.0, 0.0), jnp.float32)
        ap = zero
        for j in range(8):
            t = ai_loc[pl.ds(dl * D + j * 16, 16)] + buf[e, pl.ds(D + j * 16, 16)]
            t = jnp.maximum(t, 0.1 * t)
            ap = ap + t * w2_v[pl.ds(j * 16, 16)]
        ev = jnp.exp(jnp.full((16,), jnp.sum(ap), jnp.float32)) * m
        att2, ssum2, smax2 = [], [], []
        for j in range(8):
            hj = buf[e, pl.ds(j * 16, 16)]
            att2.append(att[j] + ev * hj)
            ssum2.append(ssum[j] + hj * m)
            smax2.append(jnp.maximum(smax[j], hj * m + (m - 1.0) * (-NEG)))
        cnt = cnt + jnp.where(act, jnp.int32(1), jnp.int32(0))
        return (cur, cnt, s_vec + ev, tuple(att2), tuple(ssum2), tuple(smax2), g0)

    def batch_step(bi, carry):
        cur, cnt, s_vec, att, ssum, smax, _ = carry
        g0 = jnp.int32((b0 + bi) * B)
        pltpu.sync_copy(srcs_hbm.at[pl.ds(g0, B)], idx_v)
        pltpu.sync_copy(dsts_hbm.at[pl.ds(g0, B)], dst_v.at[pl.ds(0, B)])
        pltpu.async_copy(haj_hbm.at[idx_v], buf, sem).wait()
        return lax.fori_loop(
            0, B, edge_step, (cur, cnt, s_vec, att, ssum, smax, g0))

    init = (jnp.int32(n0), jnp.int32(0), zero,
            (zero,) * 8, (zero,) * 8, (neg,) * 8, jnp.int32(0))
    cur, cnt, s_vec, att, ssum, smax, _ = lax.fori_loop(0, nbat, batch_step, init)
    flush_upto(cur, cnt, s_vec, att, ssum, smax, n0 + NB)


def _edge_call(haj, ai, srcs, dsts, starts, w2):
    mesh = plsc.VectorSubcoreMesh(core_axis_name="c", subcore_axis_name="s")
    f = functools.partial(
        pl.kernel,
        out_type=jax.ShapeDtypeStruct((NPAD * 3 * D,), jnp.float32),
        mesh=mesh,
        compiler_params=pltpu.CompilerParams(needs_layout_passes=False),
        scratch_types=[
            pltpu.VMEM((48,), jnp.int32),
            pltpu.VMEM((D + 16,), jnp.float32),
            pltpu.VMEM((NB * D,), jnp.float32),
            pltpu.VMEM((B,), jnp.int32),
            pltpu.VMEM((B + 16,), jnp.int32),
            pltpu.VMEM((B, 2 * D), jnp.float32),
            pltpu.VMEM((3 * D,), jnp.float32),
            pltpu.SemaphoreType.DMA,
        ],
    )(_edge_body)
    return f(haj, ai.reshape(NPAD * D), srcs, dsts, starts, w2).reshape(
        NPAD, 3 * D)


def _post_body(oc_ref, h_ref, bat_ref, g1T_ref, g1b_ref, g2T_ref, g2b_ref,
               gnw_ref, gnb_ref, gna_ref, o_ref, fus_scr, sum_scr, sq_scr,
               cnt_scr, last):
    p = pl.program_id(0)
    i = pl.program_id(1)
    rows = bat_ref[...]
    oh = (lax.broadcasted_iota(jnp.int32, (BN, GP), 1) == rows).astype(
        jnp.float32)

    @pl.when(p == 0)
    def _phase0():
        oc = oc_ref[...]
        mean_o = oc[:, :D]
        max_o = oc[:, D:2 * D]
        att_o = oc[:, 2 * D:]
        g = jnp.dot(oc, g1T_ref[...], preferred_element_type=jnp.float32)
        g = jnp.maximum(g + g1b_ref[...], 0.0)
        g2 = jnp.dot(g, g2T_ref[...], preferred_element_type=jnp.float32)
        g2 = g2 + g2b_ref[...]
        gw = jax.nn.softmax(g2, axis=-1)
        fused = (gw[:, 0:1] * mean_o + gw[:, 1:2] * max_o + gw[:, 2:3] * att_o
                 + h_ref[...])
        if last:
            o_ref[...] = fused
        else:
            fus_scr[pl.ds(i * BN, BN), :] = fused

            @pl.when(i == 0)
            def _init():
                sum_scr[...] = jnp.zeros_like(sum_scr)
                sq_scr[...] = jnp.zeros_like(sq_scr)
                cnt_scr[...] = jnp.zeros_like(cnt_scr)

            dn = (((0,), (0,)), ((), ()))
            sum_scr[...] += lax.dot_general(
                oh, fused, dn, preferred_element_type=jnp.float32)
            sq_scr[...] += lax.dot_general(
                oh, fused * fused, dn, preferred_element_type=jnp.float32)
            cnt_scr[...] += jnp.broadcast_to(
                jnp.sum(oh, axis=0)[:, None], (GP, D))

    if not last:
        @pl.when(p == 1)
        def _phase1():
            a = gna_ref[...]
            cnt = jnp.maximum(cnt_scr[...], 1.0)
            mu = sum_scr[...] / cnt
            var = sq_scr[...] / cnt - 2.0 * a * mu * mu + (a * a) * (mu * mu)
            std = jnp.sqrt(var + 1e-5)
            mu_r = jnp.dot(oh, mu, preferred_element_type=jnp.float32)
            std_r = jnp.dot(oh, std, preferred_element_type=jnp.float32)
            f = fus_scr[pl.ds(i * BN, BN), :]
            out = gnw_ref[...] * (f - a * mu_r) / std_r + gnb_ref[...]
            o_ref[...] = jnp.maximum(out, 0.0)


def _post(oc, h, bat, g1T, g1b, g2T, g2b, gnw, gnb, gna, last):
    body = functools.partial(_post_body, last=last)
    scratch = [
        pltpu.VMEM((NPAD, D), jnp.float32),
        pltpu.VMEM((GP, D), jnp.float32),
        pltpu.VMEM((GP, D), jnp.float32),
        pltpu.VMEM((GP, D), jnp.float32),
    ]
    return pl.pallas_call(
        body,
        grid=(1 if last else 2, NBLK),
        in_specs=[
            pl.BlockSpec((BN, 3 * D), lambda p, i: (i, 0)),
            pl.BlockSpec((BN, D), lambda p, i: (i, 0)),
            pl.BlockSpec((BN, 1), lambda p, i: (i, 0)),
            pl.BlockSpec((3 * D, D), lambda p, i: (0, 0)),
            pl.BlockSpec((1, D), lambda p, i: (0, 0)),
            pl.BlockSpec((D, 8), lambda p, i: (0, 0)),
            pl.BlockSpec((1, 8), lambda p, i: (0, 0)),
            pl.BlockSpec((1, D), lambda p, i: (0, 0)),
            pl.BlockSpec((1, D), lambda p, i: (0, 0)),
            pl.BlockSpec((1, D), lambda p, i: (0, 0)),
        ],
        out_specs=pl.BlockSpec((BN, D), lambda p, i: (i, 0)),
        out_shape=jax.ShapeDtypeStruct((NPAD, D), jnp.float32),
        scratch_shapes=scratch,
    )(oc, h, bat, g1T, g1b, g2T, g2b, gnw, gnb, gna)


def kernel(x, edge_index, batch, params):
    src = edge_index[0]
    dst = edge_index[1]
    perm = jnp.argsort(dst)
    srcs = src[perm].astype(jnp.int32)
    dsts = dst[perm].astype(jnp.int32)
    bounds = jnp.arange(0, NPAD + 1, NB, dtype=jnp.int32)
    starts = jnp.searchsorted(dsts, bounds).astype(jnp.int32)
    starts = jnp.concatenate(
        [starts, jnp.full((48 - starts.shape[0],), E, jnp.int32)])

    h = jnp.concatenate([x, jnp.zeros((NPAD - N, D), jnp.float32)])
    bat = jnp.concatenate(
        [batch.astype(jnp.int32), jnp.full((NPAD - N,), G, jnp.int32)])[:, None]

    for l in range(4):
        p = params
        linWT = p[f'lin_W{l}'].T
        linb = p[f'lin_b{l}'][None, :]
        wiT = p[f'att1_W{l}'][:, :D].T
        wjT = p[f'att1_W{l}'][:, D:].T
        b1 = p[f'att1_b{l}'][None, :]
        w2 = jnp.concatenate([p[f'att2_W{l}'][0], jnp.zeros((16,), jnp.float32)])
        g1T = p[f'gate1_W{l}'].T
        g1b = p[f'gate1_b{l}'][None, :]
        g2T = jnp.concatenate(
            [p[f'gate2_W{l}'].T, jnp.zeros((D, 5), jnp.float32)], axis=1)
        g2b = jnp.concatenate(
            [p[f'gate2_b{l}'], jnp.full((5,), -1e30, jnp.float32)])[None, :]

        hh, haj, ai = _pre(h, linWT, linb, wiT, wjT, b1)
        oc = _edge_call(haj, ai, srcs, dsts, starts, w2)
        last = l == 3
        if last:
            out = _post(oc, hh, bat, g1T, g1b, g2T, g2b,
                        jnp.zeros((1, D)), jnp.zeros((1, D)), jnp.zeros((1, D)),
                        True)
        else:
            h = _post(oc, hh, bat, g1T, g1b, g2T, g2b,
                      p[f'gn_w{l}'][None, :], p[f'gn_b{l}'][None, :],
                      p[f'gn_a{l}'][None, :], False)
    return out[:N]


# speedup vs baseline: 7.4308x; 1.1853x over previous
import functools

import jax
import jax.numpy as jnp
from jax import lax
from jax.experimental import pallas as pl
from jax.experimental.pallas import tpu as pltpu
from jax.experimental.pallas import tpu_sc as plsc

N = 10000
E = 320000
D = 128
G = 16

NT = 32
NB = 320
NPAD = NT * NB
B = 128
SLAB = 16
BN = 2560
NBLK = NPAD // BN
GP = 32
NEG = -1e30


def _pre_body(x_ref, linWT_ref, linb_ref, wiT_ref, wjT_ref, b1_ref,
              h_ref, haj_ref, ai_ref):
    h = jnp.dot(x_ref[...], linWT_ref[...], preferred_element_type=jnp.float32)
    h = h + linb_ref[...]
    h_ref[...] = h
    haj_ref[:, :D] = h
    haj_ref[:, D:] = jnp.dot(h, wjT_ref[...], preferred_element_type=jnp.float32)
    ai_ref[...] = (
        jnp.dot(h, wiT_ref[...], preferred_element_type=jnp.float32) + b1_ref[...]
    )


def _pre(x, linWT, linb, wiT, wjT, b1):
    return pl.pallas_call(
        _pre_body,
        grid=(NBLK,),
        in_specs=[
            pl.BlockSpec((BN, D), lambda i: (i, 0)),
            pl.BlockSpec((D, D), lambda i: (0, 0)),
            pl.BlockSpec((1, D), lambda i: (0, 0)),
            pl.BlockSpec((D, D), lambda i: (0, 0)),
            pl.BlockSpec((D, D), lambda i: (0, 0)),
            pl.BlockSpec((1, D), lambda i: (0, 0)),
        ],
        out_specs=[
            pl.BlockSpec((BN, D), lambda i: (i, 0)),
            pl.BlockSpec((BN, 2 * D), lambda i: (i, 0)),
            pl.BlockSpec((BN, D), lambda i: (i, 0)),
        ],
        out_shape=[
            jax.ShapeDtypeStruct((NPAD, D), jnp.float32),
            jax.ShapeDtypeStruct((NPAD, 2 * D), jnp.float32),
            jax.ShapeDtypeStruct((NPAD, D), jnp.float32),
        ],
    )(x, linWT, linb, wiT, wjT, b1)


def _edge_body(haj_hbm, ai_hbm, srcs_hbm, dsts_hbm, starts_hbm, w2_hbm,
               out_hbm, starts_v, w2_v, ai_loc, idx0, idx1, dst0, dst1,
               buf0, buf1, slab, semg0, semg1, semi0, semi1):
    wid = lax.axis_index("s") * 2 + lax.axis_index("c")
    n0 = wid * NB
    pltpu.sync_copy(starts_hbm, starts_v)
    pltpu.sync_copy(w2_hbm, w2_v)
    pltpu.sync_copy(ai_hbm.at[pl.ds(n0 * D, NB * D)], ai_loc)

    sv = starts_v[pl.ds(wid, 16)]
    lo = sv[0]
    hi = sv[1]
    b0 = lo // B
    nbat = jnp.where(hi > lo, (hi - b0 * B + B - 1) // B, 0)
    nhalf = (nbat + 1) // 2

    zero = jnp.zeros((16,), jnp.float32)
    neg = jnp.full((16,), NEG, jnp.float32)

    def gslice(b):
        return jnp.minimum((b0 + b) * B, E - B).astype(jnp.int32)

    def issue_idx(b, idxr, dstr, sem):
        g = gslice(b)
        pltpu.async_copy(srcs_hbm.at[pl.ds(g, B)], idxr, sem)
        pltpu.async_copy(dsts_hbm.at[pl.ds(g, B)], dstr.at[pl.ds(0, B)], sem)

    def wait_idx(idxr, dstr, sem):
        pltpu.make_async_copy(srcs_hbm.at[pl.ds(0, B)], idxr, sem).wait()
        pltpu.make_async_copy(
            dsts_hbm.at[pl.ds(0, B)], dstr.at[pl.ds(0, B)], sem).wait()

    def issue_g(idxr, bufr, sem):
        pltpu.async_copy(haj_hbm.at[idxr], bufr, sem)

    def wait_g(idxr, bufr, sem):
        pltpu.make_async_copy(haj_hbm.at[idxr], bufr, sem).wait()

    def flush_upto(cur, cnt, s_vec, att, ssum, smax, d_eff):
        cnt_f = jnp.full((16,), cnt.astype(jnp.float32), jnp.float32)
        has = jnp.minimum(cnt_f, 1.0)
        inv_c = 1.0 / jnp.maximum(cnt_f, 1.0)
        inv_s = 1.0 / jnp.maximum(s_vec, 1e-16)
        cur0 = cur

        def wr(k, _):
            mf = jnp.full((16,), jnp.where(k == cur0, 1.0, 0.0), jnp.float32)
            slot = lax.rem(k - n0, SLAB)
            base = slot * (3 * D)
            for j in range(8):
                slab[pl.ds(base + j * 16, 16)] = ssum[j] * inv_c * mf
                slab[pl.ds(base + D + j * 16, 16)] = smax[j] * has * mf
                slab[pl.ds(base + 2 * D + j * 16, 16)] = att[j] * inv_s * mf

            @pl.when(slot == SLAB - 1)
            def _out():
                pltpu.sync_copy(
                    slab,
                    out_hbm.at[pl.ds((k - (SLAB - 1)) * 3 * D, SLAB * 3 * D)])

            return 0

        lax.fori_loop(cur, d_eff, wr, 0)
        did = d_eff > cur
        kf = jnp.full((16,), jnp.where(did, 0.0, 1.0), jnp.float32)
        cnt = jnp.where(did, jnp.int32(0), cnt)
        s_vec = s_vec * kf
        att = tuple(a * kf for a in att)
        ssum = tuple(a * kf for a in ssum)
        smax = tuple(a * kf + (1.0 - kf) * NEG for a in smax)
        return jnp.maximum(cur, d_eff), cnt, s_vec, att, ssum, smax

    def compute_batch(carry, bi, bufr, dstr):
        g0 = gslice(bi)
        act_b = bi < nbat

        def edge_step(e, carry):
            cur, cnt, s_vec, att, ssum, smax = carry
            g = g0 + e
            act = jnp.logical_and(jnp.logical_and(g >= lo, g < hi), act_b)
            d = dstr[pl.ds(e, 16)][0]
            d_eff = jnp.where(act, d, cur)
            cur, cnt, s_vec, att, ssum, smax = flush_upto(
                cur, cnt, s_vec, att, ssum, smax, d_eff)
            dl = d_eff - n0
            m = jnp.full((16,), jnp.where(act, 1.0, 0.0), jnp.float32)
            ap = zero
            for j in range(8):
                t = (ai_loc[pl.ds(dl * D + j * 16, 16)]
                     + bufr[e, pl.ds(D + j * 16, 16)])
                t = jnp.maximum(t, 0.1 * t)
                ap = ap + t * w2_v[pl.ds(j * 16, 16)]
            ev = jnp.exp(jnp.full((16,), jnp.sum(ap), jnp.float32)) * m
            att2, ssum2, smax2 = [], [], []
            for j in range(8):
                hj = bufr[e, pl.ds(j * 16, 16)]
                att2.append(att[j] + ev * hj)
                ssum2.append(ssum[j] + hj * m)
                smax2.append(jnp.maximum(smax[j], hj * m + (m - 1.0) * (-NEG)))
            cnt = cnt + jnp.where(act, jnp.int32(1), jnp.int32(0))
            return (cur, cnt, s_vec + ev, tuple(att2), tuple(ssum2),
                    tuple(smax2))

        return lax.fori_loop(0, B, edge_step, carry)

    issue_idx(0, idx0, dst0, semi0)
    wait_idx(idx0, dst0, semi0)
    issue_g(idx0, buf0, semg0)
    issue_idx(1, idx1, dst1, semi1)

    def body2(i, carry):
        a = 2 * i
        wait_idx(idx1, dst1, semi1)
        issue_g(idx1, buf1, semg1)
        wait_g(idx0, buf0, semg0)
        carry = compute_batch(carry, a, buf0, dst0)
        issue_idx(a + 2, idx0, dst0, semi0)
        wait_idx(idx0, dst0, semi0)
        issue_g(idx0, buf0, semg0)
        wait_g(idx1, buf1, semg1)
        carry = compute_batch(carry, a + 1, buf1, dst1)
        issue_idx(a + 3, idx1, dst1, semi1)
        return carry

    init = (jnp.int32(n0), jnp.int32(0), zero,
            (zero,) * 8, (zero,) * 8, (neg,) * 8)
    carry = lax.fori_loop(0, nhalf, body2, init)
    wait_g(idx0, buf0, semg0)
    wait_idx(idx1, dst1, semi1)
    cur, cnt, s_vec, att, ssum, smax = carry
    flush_upto(cur, cnt, s_vec, att, ssum, smax, n0 + NB)


def _edge_call(haj, ai, srcs, dsts, starts, w2):
    mesh = plsc.VectorSubcoreMesh(core_axis_name="c", subcore_axis_name="s")
    f = functools.partial(
        pl.kernel,
        out_type=jax.ShapeDtypeStruct((NPAD * 3 * D,), jnp.float32),
        mesh=mesh,
        compiler_params=pltpu.CompilerParams(needs_layout_passes=False),
        scratch_types=[
            pltpu.VMEM((48,), jnp.int32),
            pltpu.VMEM((D + 16,), jnp.float32),
            pltpu.VMEM((NB * D,), jnp.float32),
            pltpu.VMEM((B,), jnp.int32),
            pltpu.VMEM((B,), jnp.int32),
            pltpu.VMEM((B + 16,), jnp.int32),
            pltpu.VMEM((B + 16,), jnp.int32),
            pltpu.VMEM((B, 2 * D), jnp.float32),
            pltpu.VMEM((B, 2 * D), jnp.float32),
            pltpu.VMEM((SLAB * 3 * D,), jnp.float32),
            pltpu.SemaphoreType.DMA,
            pltpu.SemaphoreType.DMA,
            pltpu.SemaphoreType.DMA,
            pltpu.SemaphoreType.DMA,
        ],
    )(_edge_body)
    return f(haj, ai.reshape(NPAD * D), srcs, dsts, starts, w2).reshape(
        NPAD, 3 * D)


def _post_body(oc_ref, h_ref, bat_ref, g1T_ref, g1b_ref, g2T_ref, g2b_ref,
               gnw_ref, gnb_ref, gna_ref, o_ref, fus_scr, sum_scr, sq_scr,
               cnt_scr, last):
    p = pl.program_id(0)
    i = pl.program_id(1)
    rows = bat_ref[...]
    oh = (lax.broadcasted_iota(jnp.int32, (BN, GP), 1) == rows).astype(
        jnp.float32)

    @pl.when(p == 0)
    def _phase0():
        oc = oc_ref[...]
        mean_o = oc[:, :D]
        max_o = oc[:, D:2 * D]
        att_o = oc[:, 2 * D:]
        g = jnp.dot(oc, g1T_ref[...], preferred_element_type=jnp.float32)
        g = jnp.maximum(g + g1b_ref[...], 0.0)
        g2 = jnp.dot(g, g2T_ref[...], preferred_element_type=jnp.float32)
        g2 = g2 + g2b_ref[...]
        gw = jax.nn.softmax(g2, axis=-1)
        fused = (gw[:, 0:1] * mean_o + gw[:, 1:2] * max_o + gw[:, 2:3] * att_o
                 + h_ref[...])
        if last:
            o_ref[...] = fused
        else:
            fus_scr[pl.ds(i * BN, BN), :] = fused

            @pl.when(i == 0)
            def _init():
                sum_scr[...] = jnp.zeros_like(sum_scr)
                sq_scr[...] = jnp.zeros_like(sq_scr)
                cnt_scr[...] = jnp.zeros_like(cnt_scr)

            dn = (((0,), (0,)), ((), ()))
            sum_scr[...] += lax.dot_general(
                oh, fused, dn, preferred_element_type=jnp.float32)
            sq_scr[...] += lax.dot_general(
                oh, fused * fused, dn, preferred_element_type=jnp.float32)
            cnt_scr[...] += jnp.broadcast_to(
                jnp.sum(oh, axis=0)[:, None], (GP, D))

    if not last:
        @pl.when(p == 1)
        def _phase1():
            a = gna_ref[...]
            cnt = jnp.maximum(cnt_scr[...], 1.0)
            mu = sum_scr[...] / cnt
            var = sq_scr[...] / cnt - 2.0 * a * mu * mu + (a * a) * (mu * mu)
            std = jnp.sqrt(var + 1e-5)
            mu_r = jnp.dot(oh, mu, preferred_element_type=jnp.float32)
            std_r = jnp.dot(oh, std, preferred_element_type=jnp.float32)
            f = fus_scr[pl.ds(i * BN, BN), :]
            out = gnw_ref[...] * (f - a * mu_r) / std_r + gnb_ref[...]
            o_ref[...] = jnp.maximum(out, 0.0)


def _post(oc, h, bat, g1T, g1b, g2T, g2b, gnw, gnb, gna, last):
    body = functools.partial(_post_body, last=last)
    scratch = [
        pltpu.VMEM((NPAD, D), jnp.float32),
        pltpu.VMEM((GP, D), jnp.float32),
        pltpu.VMEM((GP, D), jnp.float32),
        pltpu.VMEM((GP, D), jnp.float32),
    ]
    return pl.pallas_call(
        body,
        grid=(1 if last else 2, NBLK),
        in_specs=[
            pl.BlockSpec((BN, 3 * D), lambda p, i: (i, 0)),
            pl.BlockSpec((BN, D), lambda p, i: (i, 0)),
            pl.BlockSpec((BN, 1), lambda p, i: (i, 0)),
            pl.BlockSpec((3 * D, D), lambda p, i: (0, 0)),
            pl.BlockSpec((1, D), lambda p, i: (0, 0)),
            pl.BlockSpec((D, 8), lambda p, i: (0, 0)),
            pl.BlockSpec((1, 8), lambda p, i: (0, 0)),
            pl.BlockSpec((1, D), lambda p, i: (0, 0)),
            pl.BlockSpec((1, D), lambda p, i: (0, 0)),
            pl.BlockSpec((1, D), lambda p, i: (0, 0)),
        ],
        out_specs=pl.BlockSpec((BN, D), lambda p, i: (i, 0)),
        out_shape=jax.ShapeDtypeStruct((NPAD, D), jnp.float32),
        scratch_shapes=scratch,
    )(oc, h, bat, g1T, g1b, g2T, g2b, gnw, gnb, gna)


def kernel(x, edge_index, batch, params):
    src = edge_index[0]
    dst = edge_index[1]
    perm = jnp.argsort(dst)
    srcs = src[perm].astype(jnp.int32)
    dsts = dst[perm].astype(jnp.int32)
    bounds = jnp.arange(0, NPAD + 1, NB, dtype=jnp.int32)
    starts = jnp.searchsorted(dsts, bounds).astype(jnp.int32)
    starts = jnp.concatenate(
        [starts, jnp.full((48 - starts.shape[0],), E, jnp.int32)])

    h = jnp.concatenate([x, jnp.zeros((NPAD - N, D), jnp.float32)])
    bat = jnp.concatenate(
        [batch.astype(jnp.int32), jnp.full((NPAD - N,), G, jnp.int32)])[:, None]

    for l in range(4):
        p = params
        linWT = p[f'lin_W{l}'].T
        linb = p[f'lin_b{l}'][None, :]
        wiT = p[f'att1_W{l}'][:, :D].T
        wjT = p[f'att1_W{l}'][:, D:].T
        b1 = p[f'att1_b{l}'][None, :]
        w2 = jnp.concatenate([p[f'att2_W{l}'][0], jnp.zeros((16,), jnp.float32)])
        g1T = p[f'gate1_W{l}'].T
        g1b = p[f'gate1_b{l}'][None, :]
        g2T = jnp.concatenate(
            [p[f'gate2_W{l}'].T, jnp.zeros((D, 5), jnp.float32)], axis=1)
        g2b = jnp.concatenate(
            [p[f'gate2_b{l}'], jnp.full((5,), -1e30, jnp.float32)])[None, :]

        hh, haj, ai = _pre(h, linWT, linb, wiT, wjT, b1)
        oc = _edge_call(haj, ai, srcs, dsts, starts, w2)
        last = l == 3
        if last:
            out = _post(oc, hh, bat, g1T, g1b, g2T, g2b,
                        jnp.zeros((1, D)), jnp.zeros((1, D)), jnp.zeros((1, D)),
                        True)
        else:
            h = _post(oc, hh, bat, g1T, g1b, g2T, g2b,
                      p[f'gn_w{l}'][None, :], p[f'gn_b{l}'][None, :],
                      p[f'gn_a{l}'][None, :], False)
    return out[:N]


# speedup vs baseline: 7.9992x; 1.0765x over previous
import functools

import jax
import jax.numpy as jnp
from jax import lax
from jax.experimental import pallas as pl
from jax.experimental.pallas import tpu as pltpu
from jax.experimental.pallas import tpu_sc as plsc

N = 10000
E = 320000
D = 128
G = 16

NT = 32
NB = 320
NPAD = NT * NB
B = 128
SLAB = 16
BN = 2560
NBLK = NPAD // BN
GP = 32
NEG = -1e30


def _pre_body(x_ref, linWT_ref, linb_ref, wiT_ref, wjT_ref, b1_ref,
              h_ref, haj_ref, ai_ref):
    h = jnp.dot(x_ref[...], linWT_ref[...], preferred_element_type=jnp.float32)
    h = h + linb_ref[...]
    h_ref[...] = h
    haj_ref[:, :D] = h
    haj_ref[:, D:] = jnp.dot(h, wjT_ref[...], preferred_element_type=jnp.float32)
    ai_ref[...] = (
        jnp.dot(h, wiT_ref[...], preferred_element_type=jnp.float32) + b1_ref[...]
    )


def _pre(x, linWT, linb, wiT, wjT, b1):
    return pl.pallas_call(
        _pre_body,
        grid=(NBLK,),
        in_specs=[
            pl.BlockSpec((BN, D), lambda i: (i, 0)),
            pl.BlockSpec((D, D), lambda i: (0, 0)),
            pl.BlockSpec((1, D), lambda i: (0, 0)),
            pl.BlockSpec((D, D), lambda i: (0, 0)),
            pl.BlockSpec((D, D), lambda i: (0, 0)),
            pl.BlockSpec((1, D), lambda i: (0, 0)),
        ],
        out_specs=[
            pl.BlockSpec((BN, D), lambda i: (i, 0)),
            pl.BlockSpec((BN, 2 * D), lambda i: (i, 0)),
            pl.BlockSpec((BN, D), lambda i: (i, 0)),
        ],
        out_shape=[
            jax.ShapeDtypeStruct((NPAD, D), jnp.float32),
            jax.ShapeDtypeStruct((NPAD, 2 * D), jnp.float32),
            jax.ShapeDtypeStruct((NPAD, D), jnp.float32),
        ],
    )(x, linWT, linb, wiT, wjT, b1)


def _edge_body(haj_hbm, ai_hbm, srcs_hbm, dsts_hbm, starts_hbm, w2_hbm,
               out_hbm, starts_v, w2_v, ai_loc, idx0, idx1, dst0, dst1,
               buf0, buf1, slab, semg0, semg1, semi0, semi1):
    wid = lax.axis_index("s") * 2 + lax.axis_index("c")
    n0 = wid * NB
    pltpu.sync_copy(starts_hbm, starts_v)
    pltpu.sync_copy(w2_hbm, w2_v)
    pltpu.sync_copy(ai_hbm.at[pl.ds(n0 * D, NB * D)], ai_loc)

    sv = starts_v[pl.ds(wid, 16)]
    lo = sv[0]
    hi = sv[1]
    b0 = lo // B
    nbat = jnp.where(hi > lo, (hi - b0 * B + B - 1) // B, 0)
    nhalf = (nbat + 1) // 2

    zero = jnp.zeros((16,), jnp.float32)
    neg = jnp.full((16,), NEG, jnp.float32)

    def gslice(b):
        return jnp.minimum((b0 + b) * B, E - B).astype(jnp.int32)

    def issue_idx(b, idxr, dstr, sem):
        g = gslice(b)
        pltpu.async_copy(srcs_hbm.at[pl.ds(g, B)], idxr, sem)
        pltpu.async_copy(dsts_hbm.at[pl.ds(g, B)], dstr.at[pl.ds(0, B)], sem)

    def wait_idx(idxr, dstr, sem):
        pltpu.make_async_copy(srcs_hbm.at[pl.ds(0, B)], idxr, sem).wait()
        pltpu.make_async_copy(
            dsts_hbm.at[pl.ds(0, B)], dstr.at[pl.ds(0, B)], sem).wait()

    def issue_g(idxr, bufr, sem):
        pltpu.async_copy(haj_hbm.at[idxr], bufr, sem)

    def wait_g(idxr, bufr, sem):
        pltpu.make_async_copy(haj_hbm.at[idxr], bufr, sem).wait()

    def flush_upto(cur, cnt, s_vec, att, ssum, smax, d_eff):
        cnt_f = jnp.full((16,), cnt.astype(jnp.float32), jnp.float32)
        has = jnp.minimum(cnt_f, 1.0)
        inv_c = 1.0 / jnp.maximum(cnt_f, 1.0)
        inv_s = 1.0 / jnp.maximum(s_vec, 1e-16)
        cur0 = cur

        def wr(k, _):
            mf = jnp.full((16,), jnp.where(k == cur0, 1.0, 0.0), jnp.float32)
            slot = lax.rem(k - n0, SLAB)
            base = slot * (3 * D)
            for j in range(8):
                slab[pl.ds(base + j * 16, 16)] = ssum[j] * inv_c * mf
                slab[pl.ds(base + D + j * 16, 16)] = smax[j] * has * mf
                slab[pl.ds(base + 2 * D + j * 16, 16)] = att[j] * inv_s * mf

            @pl.when(slot == SLAB - 1)
            def _out():
                pltpu.sync_copy(
                    slab,
                    out_hbm.at[pl.ds((k - (SLAB - 1)) * 3 * D, SLAB * 3 * D)])

            return 0

        lax.fori_loop(cur, d_eff, wr, 0)
        did = d_eff > cur
        kf = jnp.full((16,), jnp.where(did, 0.0, 1.0), jnp.float32)
        cnt = jnp.where(did, jnp.int32(0), cnt)
        s_vec = s_vec * kf
        att = tuple(a * kf for a in att)
        ssum = tuple(a * kf for a in ssum)
        smax = tuple(a * kf + (1.0 - kf) * NEG for a in smax)
        return jnp.maximum(cur, d_eff), cnt, s_vec, att, ssum, smax

    def compute_batch(carry, bi, bufr, dstr):
        g0 = gslice(bi)
        act_b = bi < nbat
        w2c = [w2_v[pl.ds(j * 16, 16)] for j in range(8)]

        def edge_step(e, carry):
            cur, cnt, s_vec, att, ssum, smax = carry
            g = g0 + e
            act = jnp.logical_and(jnp.logical_and(g >= lo, g < hi), act_b)
            d = dstr[pl.ds(e, 16)][0]
            d_eff = jnp.where(act, d, cur)
            cur, cnt, s_vec, att, ssum, smax = flush_upto(
                cur, cnt, s_vec, att, ssum, smax, d_eff)
            dl = d_eff - n0
            m = jnp.full((16,), jnp.where(act, 1.0, 0.0), jnp.float32)
            ap = zero
            for j in range(8):
                t = (ai_loc[pl.ds(dl * D + j * 16, 16)]
                     + bufr[e, pl.ds(D + j * 16, 16)])
                t = jnp.maximum(t, 0.1 * t)
                ap = ap + t * w2c[j]
            ev = jnp.exp(jnp.full((16,), jnp.sum(ap), jnp.float32)) * m
            att2, ssum2, smax2 = [], [], []
            for j in range(8):
                hj = bufr[e, pl.ds(j * 16, 16)]
                att2.append(att[j] + ev * hj)
                ssum2.append(ssum[j] + hj * m)
                smax2.append(jnp.maximum(smax[j], hj * m + (m - 1.0) * (-NEG)))
            cnt = cnt + jnp.where(act, jnp.int32(1), jnp.int32(0))
            return (cur, cnt, s_vec + ev, tuple(att2), tuple(ssum2),
                    tuple(smax2))

        def edge2(i, c):
            return edge_step(2 * i + 1, edge_step(2 * i, c))

        return lax.fori_loop(0, B // 2, edge2, carry)

    issue_idx(0, idx0, dst0, semi0)
    wait_idx(idx0, dst0, semi0)
    issue_g(idx0, buf0, semg0)
    issue_idx(1, idx1, dst1, semi1)

    def body2(i, carry):
        a = 2 * i
        wait_idx(idx1, dst1, semi1)
        issue_g(idx1, buf1, semg1)
        wait_g(idx0, buf0, semg0)
        carry = compute_batch(carry, a, buf0, dst0)
        issue_idx(a + 2, idx0, dst0, semi0)
        wait_idx(idx0, dst0, semi0)
        issue_g(idx0, buf0, semg0)
        wait_g(idx1, buf1, semg1)
        carry = compute_batch(carry, a + 1, buf1, dst1)
        issue_idx(a + 3, idx1, dst1, semi1)
        return carry

    init = (jnp.int32(n0), jnp.int32(0), zero,
            (zero,) * 8, (zero,) * 8, (neg,) * 8)
    carry = lax.fori_loop(0, nhalf, body2, init)
    wait_g(idx0, buf0, semg0)
    wait_idx(idx1, dst1, semi1)
    cur, cnt, s_vec, att, ssum, smax = carry
    flush_upto(cur, cnt, s_vec, att, ssum, smax, n0 + NB)


def _edge_call(haj, ai, srcs, dsts, starts, w2):
    mesh = plsc.VectorSubcoreMesh(core_axis_name="c", subcore_axis_name="s")
    f = functools.partial(
        pl.kernel,
        out_type=jax.ShapeDtypeStruct((NPAD * 3 * D,), jnp.float32),
        mesh=mesh,
        compiler_params=pltpu.CompilerParams(needs_layout_passes=False),
        scratch_types=[
            pltpu.VMEM((48,), jnp.int32),
            pltpu.VMEM((D + 16,), jnp.float32),
            pltpu.VMEM((NB * D,), jnp.float32),
            pltpu.VMEM((B,), jnp.int32),
            pltpu.VMEM((B,), jnp.int32),
            pltpu.VMEM((B + 16,), jnp.int32),
            pltpu.VMEM((B + 16,), jnp.int32),
            pltpu.VMEM((B, 2 * D), jnp.float32),
            pltpu.VMEM((B, 2 * D), jnp.float32),
            pltpu.VMEM((SLAB * 3 * D,), jnp.float32),
            pltpu.SemaphoreType.DMA,
            pltpu.SemaphoreType.DMA,
            pltpu.SemaphoreType.DMA,
            pltpu.SemaphoreType.DMA,
        ],
    )(_edge_body)
    return f(haj, ai.reshape(NPAD * D), srcs, dsts, starts, w2).reshape(
        NPAD, 3 * D)


def _post_body(oc_ref, h_ref, bat_ref, g1T_ref, g1b_ref, g2T_ref, g2b_ref,
               gnw_ref, gnb_ref, gna_ref, o_ref, fus_scr, sum_scr, sq_scr,
               cnt_scr, last):
    p = pl.program_id(0)
    i = pl.program_id(1)
    rows = bat_ref[...]
    oh = (lax.broadcasted_iota(jnp.int32, (BN, GP), 1) == rows).astype(
        jnp.float32)

    @pl.when(p == 0)
    def _phase0():
        oc = oc_ref[...]
        mean_o = oc[:, :D]
        max_o = oc[:, D:2 * D]
        att_o = oc[:, 2 * D:]
        g = jnp.dot(oc, g1T_ref[...], preferred_element_type=jnp.float32)
        g = jnp.maximum(g + g1b_ref[...], 0.0)
        g2 = jnp.dot(g, g2T_ref[...], preferred_element_type=jnp.float32)
        g2 = g2 + g2b_ref[...]
        gw = jax.nn.softmax(g2, axis=-1)
        fused = (gw[:, 0:1] * mean_o + gw[:, 1:2] * max_o + gw[:, 2:3] * att_o
                 + h_ref[...])
        if last:
            o_ref[...] = fused
        else:
            fus_scr[pl.ds(i * BN, BN), :] = fused

            @pl.when(i == 0)
            def _init():
                sum_scr[...] = jnp.zeros_like(sum_scr)
                sq_scr[...] = jnp.zeros_like(sq_scr)
                cnt_scr[...] = jnp.zeros_like(cnt_scr)

            dn = (((0,), (0,)), ((), ()))
            sum_scr[...] += lax.dot_general(
                oh, fused, dn, preferred_element_type=jnp.float32)
            sq_scr[...] += lax.dot_general(
                oh, fused * fused, dn, preferred_element_type=jnp.float32)
            cnt_scr[...] += jnp.broadcast_to(
                jnp.sum(oh, axis=0)[:, None], (GP, D))

    if not last:
        @pl.when(p == 1)
        def _phase1():
            a = gna_ref[...]
            cnt = jnp.maximum(cnt_scr[...], 1.0)
            mu = sum_scr[...] / cnt
            var = sq_scr[...] / cnt - 2.0 * a * mu * mu + (a * a) * (mu * mu)
            std = jnp.sqrt(var + 1e-5)
            mu_r = jnp.dot(oh, mu, preferred_element_type=jnp.float32)
            std_r = jnp.dot(oh, std, preferred_element_type=jnp.float32)
            f = fus_scr[pl.ds(i * BN, BN), :]
            out = gnw_ref[...] * (f - a * mu_r) / std_r + gnb_ref[...]
            o_ref[...] = jnp.maximum(out, 0.0)


def _post(oc, h, bat, g1T, g1b, g2T, g2b, gnw, gnb, gna, last):
    body = functools.partial(_post_body, last=last)
    scratch = [
        pltpu.VMEM((NPAD, D), jnp.float32),
        pltpu.VMEM((GP, D), jnp.float32),
        pltpu.VMEM((GP, D), jnp.float32),
        pltpu.VMEM((GP, D), jnp.float32),
    ]
    return pl.pallas_call(
        body,
        grid=(1 if last else 2, NBLK),
        in_specs=[
            pl.BlockSpec((BN, 3 * D), lambda p, i: (i, 0)),
            pl.BlockSpec((BN, D), lambda p, i: (i, 0)),
            pl.BlockSpec((BN, 1), lambda p, i: (i, 0)),
            pl.BlockSpec((3 * D, D), lambda p, i: (0, 0)),
            pl.BlockSpec((1, D), lambda p, i: (0, 0)),
            pl.BlockSpec((D, 8), lambda p, i: (0, 0)),
            pl.BlockSpec((1, 8), lambda p, i: (0, 0)),
            pl.BlockSpec((1, D), lambda p, i: (0, 0)),
            pl.BlockSpec((1, D), lambda p, i: (0, 0)),
            pl.BlockSpec((1, D), lambda p, i: (0, 0)),
        ],
        out_specs=pl.BlockSpec((BN, D), lambda p, i: (i, 0)),
        out_shape=jax.ShapeDtypeStruct((NPAD, D), jnp.float32),
        scratch_shapes=scratch,
    )(oc, h, bat, g1T, g1b, g2T, g2b, gnw, gnb, gna)


def kernel(x, edge_index, batch, params):
    src = edge_index[0]
    dst = edge_index[1]
    perm = jnp.argsort(dst)
    srcs = src[perm].astype(jnp.int32)
    dsts = dst[perm].astype(jnp.int32)
    bounds = jnp.arange(0, NPAD + 1, NB, dtype=jnp.int32)
    starts = jnp.searchsorted(dsts, bounds).astype(jnp.int32)
    starts = jnp.concatenate(
        [starts, jnp.full((48 - starts.shape[0],), E, jnp.int32)])

    h = jnp.concatenate([x, jnp.zeros((NPAD - N, D), jnp.float32)])
    bat = jnp.concatenate(
        [batch.astype(jnp.int32), jnp.full((NPAD - N,), G, jnp.int32)])[:, None]

    for l in range(4):
        p = params
        linWT = p[f'lin_W{l}'].T
        linb = p[f'lin_b{l}'][None, :]
        wiT = p[f'att1_W{l}'][:, :D].T
        wjT = p[f'att1_W{l}'][:, D:].T
        b1 = p[f'att1_b{l}'][None, :]
        w2 = jnp.concatenate([p[f'att2_W{l}'][0], jnp.zeros((16,), jnp.float32)])
        g1T = p[f'gate1_W{l}'].T
        g1b = p[f'gate1_b{l}'][None, :]
        g2T = jnp.concatenate(
            [p[f'gate2_W{l}'].T, jnp.zeros((D, 5), jnp.float32)], axis=1)
        g2b = jnp.concatenate(
            [p[f'gate2_b{l}'], jnp.full((5,), -1e30, jnp.float32)])[None, :]

        hh, haj, ai = _pre(h, linWT, linb, wiT, wjT, b1)
        oc = _edge_call(haj, ai, srcs, dsts, starts, w2)
        last = l == 3
        if last:
            out = _post(oc, hh, bat, g1T, g1b, g2T, g2b,
                        jnp.zeros((1, D)), jnp.zeros((1, D)), jnp.zeros((1, D)),
                        True)
        else:
            h = _post(oc, hh, bat, g1T, g1b, g2T, g2b,
                      p[f'gn_w{l}'][None, :], p[f'gn_b{l}'][None, :],
                      p[f'gn_a{l}'][None, :], False)
    return out[:N]


# speedup vs baseline: 8.4761x; 1.0596x over previous
import functools

import jax
import jax.numpy as jnp
from jax import lax
from jax.experimental import pallas as pl
from jax.experimental.pallas import tpu as pltpu
from jax.experimental.pallas import tpu_sc as plsc

N = 10000
E = 320000
D = 128
G = 16

NT = 32
NB = 320
NPAD = NT * NB
B = 128
SLAB = 16
BN = 2560
NBLK = NPAD // BN
GP = 32
NEG = -1e30


def _pre_body(x_ref, linWT_ref, linb_ref, wiT_ref, wjT_ref, b1_ref,
              h_ref, haj_ref, ai_ref):
    h = jnp.dot(x_ref[...], linWT_ref[...], preferred_element_type=jnp.float32)
    h = h + linb_ref[...]
    h_ref[...] = h
    haj_ref[:, :D] = h
    haj_ref[:, D:] = jnp.dot(h, wjT_ref[...], preferred_element_type=jnp.float32)
    ai_ref[...] = (
        jnp.dot(h, wiT_ref[...], preferred_element_type=jnp.float32) + b1_ref[...]
    )


def _pre(x, linWT, linb, wiT, wjT, b1):
    return pl.pallas_call(
        _pre_body,
        grid=(NBLK,),
        in_specs=[
            pl.BlockSpec((BN, D), lambda i: (i, 0)),
            pl.BlockSpec((D, D), lambda i: (0, 0)),
            pl.BlockSpec((1, D), lambda i: (0, 0)),
            pl.BlockSpec((D, D), lambda i: (0, 0)),
            pl.BlockSpec((D, D), lambda i: (0, 0)),
            pl.BlockSpec((1, D), lambda i: (0, 0)),
        ],
        out_specs=[
            pl.BlockSpec((BN, D), lambda i: (i, 0)),
            pl.BlockSpec((BN, 2 * D), lambda i: (i, 0)),
            pl.BlockSpec((BN, D), lambda i: (i, 0)),
        ],
        out_shape=[
            jax.ShapeDtypeStruct((NPAD, D), jnp.float32),
            jax.ShapeDtypeStruct((NPAD, 2 * D), jnp.float32),
            jax.ShapeDtypeStruct((NPAD, D), jnp.float32),
        ],
    )(x, linWT, linb, wiT, wjT, b1)


def _edge_body(haj_hbm, ai_hbm, srcs_hbm, dsts_hbm, starts_hbm, w2_hbm,
               out_hbm, starts_v, w2_v, ai_loc, idx0, idx1, dst0, dst1,
               buf0, buf1, slab, semg0, semg1, semi0, semi1):
    wid = lax.axis_index("s") * 2 + lax.axis_index("c")
    n0 = wid * NB
    pltpu.sync_copy(starts_hbm, starts_v)
    pltpu.sync_copy(w2_hbm, w2_v)
    pltpu.sync_copy(ai_hbm.at[pl.ds(n0 * D, NB * D)], ai_loc)

    sv = starts_v[pl.ds(wid, 16)]
    lo = sv[0]
    hi = sv[1]
    b0 = lo // B
    nbat = jnp.where(hi > lo, (hi - b0 * B + B - 1) // B, 0)
    nhalf = (nbat + 1) // 2

    zero = jnp.zeros((16,), jnp.float32)
    neg = jnp.full((16,), NEG, jnp.float32)

    def gslice(b):
        return jnp.minimum((b0 + b) * B, E - B).astype(jnp.int32)

    def issue_idx(b, idxr, dstr, sem):
        g = gslice(b)
        pltpu.async_copy(srcs_hbm.at[pl.ds(g, B)], idxr, sem)
        pltpu.async_copy(dsts_hbm.at[pl.ds(g, B)], dstr.at[pl.ds(0, B)], sem)

    def wait_idx(idxr, dstr, sem):
        pltpu.make_async_copy(srcs_hbm.at[pl.ds(0, B)], idxr, sem).wait()
        pltpu.make_async_copy(
            dsts_hbm.at[pl.ds(0, B)], dstr.at[pl.ds(0, B)], sem).wait()

    def issue_g(idxr, bufr, sem):
        pltpu.async_copy(haj_hbm.at[idxr], bufr, sem)

    def wait_g(idxr, bufr, sem):
        pltpu.make_async_copy(haj_hbm.at[idxr], bufr, sem).wait()

    def flush_upto(cur, cnt, s_vec, att, ssum, smax, d_eff):
        cnt_f = jnp.full((16,), cnt.astype(jnp.float32), jnp.float32)
        has = jnp.minimum(cnt_f, 1.0)
        inv_c = 1.0 / jnp.maximum(cnt_f, 1.0)
        inv_s = 1.0 / jnp.maximum(s_vec, 1e-16)
        cur0 = cur

        def wr(k, _):
            mf = jnp.full((16,), jnp.where(k == cur0, 1.0, 0.0), jnp.float32)
            slot = lax.rem(k - n0, SLAB)
            base = slot * (3 * D)
            for j in range(8):
                slab[pl.ds(base + j * 16, 16)] = ssum[j] * inv_c * mf
                slab[pl.ds(base + D + j * 16, 16)] = smax[j] * has * mf
                slab[pl.ds(base + 2 * D + j * 16, 16)] = att[j] * inv_s * mf

            @pl.when(slot == SLAB - 1)
            def _out():
                pltpu.sync_copy(
                    slab,
                    out_hbm.at[pl.ds((k - (SLAB - 1)) * 3 * D, SLAB * 3 * D)])

            return 0

        lax.fori_loop(cur, d_eff, wr, 0)
        did = d_eff > cur
        kf = jnp.full((16,), jnp.where(did, 0.0, 1.0), jnp.float32)
        cnt = jnp.where(did, jnp.int32(0), cnt)
        s_vec = s_vec * kf
        att = tuple(a * kf for a in att)
        ssum = tuple(a * kf for a in ssum)
        smax = tuple(a * kf + (1.0 - kf) * NEG for a in smax)
        return jnp.maximum(cur, d_eff), cnt, s_vec, att, ssum, smax

    def compute_batch(carry, bi, bufr, dstr):
        g0 = gslice(bi)
        act_b = bi < nbat
        w2c = [w2_v[pl.ds(j * 16, 16)] for j in range(8)]

        def edge_step(e, carry):
            cur, cnt, s_vec, att, ssum, smax = carry
            g = g0 + e
            act = jnp.logical_and(jnp.logical_and(g >= lo, g < hi), act_b)
            d = dstr[pl.ds(e, 16)][0]
            d_eff = jnp.where(act, d, cur)
            cur, cnt, s_vec, att, ssum, smax = flush_upto(
                cur, cnt, s_vec, att, ssum, smax, d_eff)
            dl = d_eff - n0
            m = jnp.full((16,), jnp.where(act, 1.0, 0.0), jnp.float32)
            ap = zero
            for j in range(8):
                t = (ai_loc[pl.ds(dl * D + j * 16, 16)]
                     + bufr[e, pl.ds(D + j * 16, 16)])
                t = jnp.maximum(t, 0.1 * t)
                ap = ap + t * w2c[j]
            ev = jnp.exp(jnp.full((16,), jnp.sum(ap), jnp.float32)) * m
            att2, ssum2, smax2 = [], [], []
            for j in range(8):
                hj = bufr[e, pl.ds(j * 16, 16)]
                att2.append(att[j] + ev * hj)
                ssum2.append(ssum[j] + hj * m)
                smax2.append(jnp.maximum(smax[j], hj * m + (m - 1.0) * (-NEG)))
            cnt = cnt + jnp.where(act, jnp.int32(1), jnp.int32(0))
            return (cur, cnt, s_vec + ev, tuple(att2), tuple(ssum2),
                    tuple(smax2))

        def edge4(i, c):
            c = edge_step(4 * i + 1, edge_step(4 * i, c))
            return edge_step(4 * i + 3, edge_step(4 * i + 2, c))

        return lax.fori_loop(0, B // 4, edge4, carry)

    issue_idx(0, idx0, dst0, semi0)
    wait_idx(idx0, dst0, semi0)
    issue_g(idx0, buf0, semg0)
    issue_idx(1, idx1, dst1, semi1)

    def body2(i, carry):
        a = 2 * i
        wait_idx(idx1, dst1, semi1)
        issue_g(idx1, buf1, semg1)
        wait_g(idx0, buf0, semg0)
        carry = compute_batch(carry, a, buf0, dst0)
        issue_idx(a + 2, idx0, dst0, semi0)
        wait_idx(idx0, dst0, semi0)
        issue_g(idx0, buf0, semg0)
        wait_g(idx1, buf1, semg1)
        carry = compute_batch(carry, a + 1, buf1, dst1)
        issue_idx(a + 3, idx1, dst1, semi1)
        return carry

    init = (jnp.int32(n0), jnp.int32(0), zero,
            (zero,) * 8, (zero,) * 8, (neg,) * 8)
    carry = lax.fori_loop(0, nhalf, body2, init)
    wait_g(idx0, buf0, semg0)
    wait_idx(idx1, dst1, semi1)
    cur, cnt, s_vec, att, ssum, smax = carry
    flush_upto(cur, cnt, s_vec, att, ssum, smax, n0 + NB)


def _edge_call(haj, ai, srcs, dsts, starts, w2):
    mesh = plsc.VectorSubcoreMesh(core_axis_name="c", subcore_axis_name="s")
    f = functools.partial(
        pl.kernel,
        out_type=jax.ShapeDtypeStruct((NPAD * 3 * D,), jnp.float32),
        mesh=mesh,
        compiler_params=pltpu.CompilerParams(needs_layout_passes=False),
        scratch_types=[
            pltpu.VMEM((48,), jnp.int32),
            pltpu.VMEM((D + 16,), jnp.float32),
            pltpu.VMEM((NB * D,), jnp.float32),
            pltpu.VMEM((B,), jnp.int32),
            pltpu.VMEM((B,), jnp.int32),
            pltpu.VMEM((B + 16,), jnp.int32),
            pltpu.VMEM((B + 16,), jnp.int32),
            pltpu.VMEM((B, 2 * D), jnp.float32),
            pltpu.VMEM((B, 2 * D), jnp.float32),
            pltpu.VMEM((SLAB * 3 * D,), jnp.float32),
            pltpu.SemaphoreType.DMA,
            pltpu.SemaphoreType.DMA,
            pltpu.SemaphoreType.DMA,
            pltpu.SemaphoreType.DMA,
        ],
    )(_edge_body)
    return f(haj, ai.reshape(NPAD * D), srcs, dsts, starts, w2).reshape(
        NPAD, 3 * D)


def _post_body(oc_ref, h_ref, bat_ref, g1T_ref, g1b_ref, g2T_ref, g2b_ref,
               gnw_ref, gnb_ref, gna_ref, o_ref, fus_scr, sum_scr, sq_scr,
               cnt_scr, last):
    p = pl.program_id(0)
    i = pl.program_id(1)
    rows = bat_ref[...]
    oh = (lax.broadcasted_iota(jnp.int32, (BN, GP), 1) == rows).astype(
        jnp.float32)

    @pl.when(p == 0)
    def _phase0():
        oc = oc_ref[...]
        mean_o = oc[:, :D]
        max_o = oc[:, D:2 * D]
        att_o = oc[:, 2 * D:]
        g = jnp.dot(oc, g1T_ref[...], preferred_element_type=jnp.float32)
        g = jnp.maximum(g + g1b_ref[...], 0.0)
        g2 = jnp.dot(g, g2T_ref[...], preferred_element_type=jnp.float32)
        g2 = g2 + g2b_ref[...]
        gw = jax.nn.softmax(g2, axis=-1)
        fused = (gw[:, 0:1] * mean_o + gw[:, 1:2] * max_o + gw[:, 2:3] * att_o
                 + h_ref[...])
        if last:
            o_ref[...] = fused
        else:
            fus_scr[pl.ds(i * BN, BN), :] = fused

            @pl.when(i == 0)
            def _init():
                sum_scr[...] = jnp.zeros_like(sum_scr)
                sq_scr[...] = jnp.zeros_like(sq_scr)
                cnt_scr[...] = jnp.zeros_like(cnt_scr)

            dn = (((0,), (0,)), ((), ()))
            sum_scr[...] += lax.dot_general(
                oh, fused, dn, preferred_element_type=jnp.float32)
            sq_scr[...] += lax.dot_general(
                oh, fused * fused, dn, preferred_element_type=jnp.float32)
            cnt_scr[...] += jnp.broadcast_to(
                jnp.sum(oh, axis=0)[:, None], (GP, D))

    if not last:
        @pl.when(p == 1)
        def _phase1():
            a = gna_ref[...]
            cnt = jnp.maximum(cnt_scr[...], 1.0)
            mu = sum_scr[...] / cnt
            var = sq_scr[...] / cnt - 2.0 * a * mu * mu + (a * a) * (mu * mu)
            std = jnp.sqrt(var + 1e-5)
            mu_r = jnp.dot(oh, mu, preferred_element_type=jnp.float32)
            std_r = jnp.dot(oh, std, preferred_element_type=jnp.float32)
            f = fus_scr[pl.ds(i * BN, BN), :]
            out = gnw_ref[...] * (f - a * mu_r) / std_r + gnb_ref[...]
            o_ref[...] = jnp.maximum(out, 0.0)


def _post(oc, h, bat, g1T, g1b, g2T, g2b, gnw, gnb, gna, last):
    body = functools.partial(_post_body, last=last)
    scratch = [
        pltpu.VMEM((NPAD, D), jnp.float32),
        pltpu.VMEM((GP, D), jnp.float32),
        pltpu.VMEM((GP, D), jnp.float32),
        pltpu.VMEM((GP, D), jnp.float32),
    ]
    return pl.pallas_call(
        body,
        grid=(1 if last else 2, NBLK),
        in_specs=[
            pl.BlockSpec((BN, 3 * D), lambda p, i: (i, 0)),
            pl.BlockSpec((BN, D), lambda p, i: (i, 0)),
            pl.BlockSpec((BN, 1), lambda p, i: (i, 0)),
            pl.BlockSpec((3 * D, D), lambda p, i: (0, 0)),
            pl.BlockSpec((1, D), lambda p, i: (0, 0)),
            pl.BlockSpec((D, 8), lambda p, i: (0, 0)),
            pl.BlockSpec((1, 8), lambda p, i: (0, 0)),
            pl.BlockSpec((1, D), lambda p, i: (0, 0)),
            pl.BlockSpec((1, D), lambda p, i: (0, 0)),
            pl.BlockSpec((1, D), lambda p, i: (0, 0)),
        ],
        out_specs=pl.BlockSpec((BN, D), lambda p, i: (i, 0)),
        out_shape=jax.ShapeDtypeStruct((NPAD, D), jnp.float32),
        scratch_shapes=scratch,
    )(oc, h, bat, g1T, g1b, g2T, g2b, gnw, gnb, gna)


def kernel(x, edge_index, batch, params):
    src = edge_index[0]
    dst = edge_index[1]
    perm = jnp.argsort(dst)
    srcs = src[perm].astype(jnp.int32)
    dsts = dst[perm].astype(jnp.int32)
    bounds = jnp.arange(0, NPAD + 1, NB, dtype=jnp.int32)
    starts = jnp.searchsorted(dsts, bounds).astype(jnp.int32)
    starts = jnp.concatenate(
        [starts, jnp.full((48 - starts.shape[0],), E, jnp.int32)])

    h = jnp.concatenate([x, jnp.zeros((NPAD - N, D), jnp.float32)])
    bat = jnp.concatenate(
        [batch.astype(jnp.int32), jnp.full((NPAD - N,), G, jnp.int32)])[:, None]

    for l in range(4):
        p = params
        linWT = p[f'lin_W{l}'].T
        linb = p[f'lin_b{l}'][None, :]
        wiT = p[f'att1_W{l}'][:, :D].T
        wjT = p[f'att1_W{l}'][:, D:].T
        b1 = p[f'att1_b{l}'][None, :]
        w2 = jnp.concatenate([p[f'att2_W{l}'][0], jnp.zeros((16,), jnp.float32)])
        g1T = p[f'gate1_W{l}'].T
        g1b = p[f'gate1_b{l}'][None, :]
        g2T = jnp.concatenate(
            [p[f'gate2_W{l}'].T, jnp.zeros((D, 5), jnp.float32)], axis=1)
        g2b = jnp.concatenate(
            [p[f'gate2_b{l}'], jnp.full((5,), -1e30, jnp.float32)])[None, :]

        hh, haj, ai = _pre(h, linWT, linb, wiT, wjT, b1)
        oc = _edge_call(haj, ai, srcs, dsts, starts, w2)
        last = l == 3
        if last:
            out = _post(oc, hh, bat, g1T, g1b, g2T, g2b,
                        jnp.zeros((1, D)), jnp.zeros((1, D)), jnp.zeros((1, D)),
                        True)
        else:
            h = _post(oc, hh, bat, g1T, g1b, g2T, g2b,
                      p[f'gn_w{l}'][None, :], p[f'gn_b{l}'][None, :],
                      p[f'gn_a{l}'][None, :], False)
    return out[:N]
